# baseline (speedup 1.0000x reference)
import jax
import jax.numpy as jnp
from jax.experimental import pallas as pl
from jax.experimental.pallas import tpu as pltpu

_NP = 20000
_R, _C = 160, 128
_PP = _R * _C
_NCLS = 4
_NOBJ = 50
_TH = 0.5
_NEGPOS = 3
_VAR = 0.1
_ALPHA = 0.1


def _smooth_l1(d):
    ad = jnp.abs(d)
    return jnp.where(ad < 1.0, 0.5 * ad * ad, ad - 0.5)


_SPP = 2


def _one_sample(tgt_ref, loc_ref, cnf_ref, reg_ref, bti_sa, s, pri):
    f32 = jnp.float32
    pcx, pcy, pw, ph, px0, py0, px1, py1, area_p, idx, valid = pri

    bto = jnp.full((_R, _C), -1.0, f32)
    bti = jnp.zeros((_R, _C), jnp.int32)
    bpi = []
    n_tiles = _R // 8
    for j in range(_NOBJ):
        tx0 = tgt_ref[0, s, j, 0]
        ty0 = tgt_ref[0, s, j, 1]
        tx1 = tgt_ref[0, s, j, 2]
        ty1 = tgt_ref[0, s, j, 3]
        area_t = (tx1 - tx0) * (ty1 - ty0)
        iw = jnp.maximum(jnp.minimum(tx1, px1) - jnp.maximum(tx0, px0), 0.0)
        ih = jnp.maximum(jnp.minimum(ty1, py1) - jnp.maximum(ty0, py0), 0.0)
        inter = iw * ih
        ov = inter / (area_t + area_p - inter)
        upd = ov > bto
        bto = jnp.where(upd, ov, bto)
        bti = jnp.where(upd, j, bti)
        vmax = ov[0:8, :]
        vidx = idx[0:8, :]
        for t in range(1, n_tiles):
            ot = ov[8 * t:8 * t + 8, :]
            u = ot > vmax
            vmax = jnp.where(u, ot, vmax)
            vidx = jnp.where(u, idx[8 * t:8 * t + 8, :], vidx)
        m = jnp.max(vmax)
        bpi.append(jnp.min(jnp.where(vmax == m, vidx, _PP)))

    bti_sa[s] = bti
    colv = jax.lax.broadcasted_iota(jnp.int32, (1, _C), 1)
    for j in range(_NOBJ):
        r = bpi[j] // _C
        c = bpi[j] % _C
        rowv = bti_sa[s, pl.ds(r, 1), :]
        bti_sa[s, pl.ds(r, 1), :] = jnp.where(colv == c, j + 64, rowv)
    btif = bti_sa[s]
    forced = btif >= 64
    btim = btif & 63

    mcx = jnp.zeros((_R, _C), f32)
    mcy = jnp.zeros((_R, _C), f32)
    lbl = jnp.zeros((_R, _C), f32)
    rgt = jnp.zeros((_R, _C), f32)
    for j in range(_NOBJ):
        mj = btim == j
        mcx = jnp.where(mj, (tgt_ref[0, s, j, 0] + tgt_ref[0, s, j, 2]) * 0.5, mcx)
        mcy = jnp.where(mj, (tgt_ref[0, s, j, 1] + tgt_ref[0, s, j, 3]) * 0.5, mcy)
        lbl = jnp.where(mj, tgt_ref[0, s, j, 4], lbl)
        rgt = jnp.where(mj, tgt_ref[0, s, j, 5], rgt)

    conf = jnp.where(jnp.logical_or(forced, bto >= _TH),
                     lbl.astype(jnp.int32) + 1, 0)
    pos = conf > 0
    posf = pos.astype(f32)
    num_pos = jnp.sum(posf)

    lt0 = (mcx - pcx) / (_VAR * pw)
    lt1 = (mcy - pcy) / (_VAR * ph)
    loss_l = jnp.sum((_smooth_l1(loc_ref[0, s, 0] - lt0)
                      + _smooth_l1(loc_ref[0, s, 1] - lt1)) * posf)

    loss_r = jnp.sum(_smooth_l1(reg_ref[0, s, 0] - rgt) * posf)

    c0 = cnf_ref[0, s, 0]
    c1 = cnf_ref[0, s, 1]
    c2 = cnf_ref[0, s, 2]
    c3 = cnf_ref[0, s, 3]
    cm = jnp.maximum(jnp.maximum(c0, c1), jnp.maximum(c2, c3))
    lse = cm + jnp.log(jnp.exp(c0 - cm) + jnp.exp(c1 - cm)
                       + jnp.exp(c2 - cm) + jnp.exp(c3 - cm))
    g = jnp.where(conf == 0, c0, 0.0) + jnp.where(conf == 1, c1, 0.0) \
        + jnp.where(conf == 2, c2, 0.0) + jnp.where(conf == 3, c3, 0.0)
    lca = lse - g
    lca = jnp.where(pos, 0.0, lca)
    lca = jnp.where(valid, lca, -1.0)

    bits = jax.lax.bitcast_convert_type(lca, jnp.int32)
    num_neg = jnp.minimum(_NEGPOS * num_pos.astype(jnp.int32), _NP - 1)
    thr = jnp.int32(0)
    for k in range(30, -1, -1):
        cand = thr | jnp.int32(1 << k)
        cnt = jnp.sum(jnp.where(bits >= cand, 1, 0))
        thr = jnp.where(cnt >= num_neg, cand, thr)
    neg = bits >= thr
    self32 = jnp.logical_or(pos, neg).astype(f32)

    bce = jnp.zeros((_R, _C), f32)
    for k, ck in enumerate((c0, c1, c2, c3)):
        st = jnp.where(conf == k, 1.0 - _ALPHA, 0.0) + _ALPHA / 4.0
        bce = bce + (jnp.maximum(ck, 0.0) - ck * st
                     + jnp.log1p(jnp.exp(-jnp.abs(ck))))
    loss_c = jnp.sum(bce * self32)

    return loss_l, loss_c, loss_r, num_pos


def _body(tgt_ref, loc_ref, cnf_ref, reg_ref, pri_ref,
          out_l, out_c, out_r, out_n, bti_sa):
    b = pl.program_id(0)
    f32 = jnp.float32

    pcx = pri_ref[0]
    pcy = pri_ref[1]
    pw = pri_ref[2]
    ph = pri_ref[3]
    px0 = pcx - pw * 0.5
    py0 = pcy - ph * 0.5
    px1 = pcx + pw * 0.5
    py1 = pcy + ph * 0.5
    area_p = (px1 - px0) * (py1 - py0)
    row = jax.lax.broadcasted_iota(jnp.int32, (_R, _C), 0)
    col = jax.lax.broadcasted_iota(jnp.int32, (_R, _C), 1)
    idx = row * _C + col
    valid = idx < _NP
    pri = (pcx, pcy, pw, ph, px0, py0, px1, py1, area_p, idx, valid)

    acc = [jnp.float32(0.0)] * 4
    for s in range(_SPP):
        res = _one_sample(tgt_ref, loc_ref, cnf_ref, reg_ref, bti_sa, s, pri)
        acc = [a + r for a, r in zip(acc, res)]

    @pl.when(b == 0)
    def _():
        out_l[...] = jnp.zeros((8, 128), f32)
        out_c[...] = jnp.zeros((8, 128), f32)
        out_r[...] = jnp.zeros((8, 128), f32)
        out_n[...] = jnp.zeros((8, 128), f32)

    out_l[...] = out_l[...] + acc[0]
    out_c[...] = out_c[...] + acc[1]
    out_r[...] = out_r[...] + acc[2]
    out_n[...] = out_n[...] + acc[3]


def kernel(loc_data, cnf_data, reg_data, targets, priors):
    bs = loc_data.shape[0]
    pad = _PP - _NP
    f32 = jnp.float32

    pad_pri = jnp.broadcast_to(
        jnp.array([-5.0, -5.0, 1e-3, 1e-3], f32), (pad, 4))
    pri = jnp.concatenate([priors.astype(f32), pad_pri], axis=0)
    pri = pri.T.reshape(4, _R, _C)

    def pad_t(x):
        x = jnp.pad(x, ((0, 0), (0, pad), (0, 0)))
        return x.transpose(0, 2, 1).reshape(bs, x.shape[2], _R, _C)

    loc = pad_t(loc_data)
    cnf = pad_t(cnf_data)
    reg = pad_t(reg_data)
    tgt = targets.reshape(bs, _NOBJ, 6)

    loc = loc.reshape(bs // _SPP, _SPP, 2, _R, _C)
    cnf = cnf.reshape(bs // _SPP, _SPP, _NCLS, _R, _C)
    reg = reg.reshape(bs // _SPP, _SPP, 1, _R, _C)
    tgt = tgt.reshape(bs // _SPP, _SPP, _NOBJ, 6)

    out_shape = [jax.ShapeDtypeStruct((8, 128), f32)] * 4
    outs = pl.pallas_call(
        _body,
        grid=(bs // _SPP,),
        in_specs=[
            pl.BlockSpec((1, _SPP, _NOBJ, 6), lambda b: (b, 0, 0, 0),
                         memory_space=pltpu.SMEM),
            pl.BlockSpec((1, _SPP, 2, _R, _C), lambda b: (b, 0, 0, 0, 0)),
            pl.BlockSpec((1, _SPP, _NCLS, _R, _C), lambda b: (b, 0, 0, 0, 0)),
            pl.BlockSpec((1, _SPP, 1, _R, _C), lambda b: (b, 0, 0, 0, 0)),
            pl.BlockSpec((4, _R, _C), lambda b: (0, 0, 0)),
        ],
        out_specs=[pl.BlockSpec((8, 128), lambda b: (0, 0))] * 4,
        out_shape=out_shape,
        scratch_shapes=[pltpu.VMEM((_SPP, _R, _C), jnp.int32)],
    )(tgt, loc, cnf, reg, pri)

    l, c, r, n = [o[0, 0] for o in outs]
    return (l / n, c / n, r / n)


# speedup vs baseline: 1.0725x; 1.0725x over previous
import jax
import jax.numpy as jnp
from jax.experimental import pallas as pl
from jax.experimental.pallas import tpu as pltpu

_NP = 20000
_R, _C = 160, 128
_PP = _R * _C
_NCLS = 4
_NOBJ = 50
_TH = 0.5
_NEGPOS = 3
_VAR = 0.1
_ALPHA = 0.1


def _smooth_l1(d):
    ad = jnp.abs(d)
    return jnp.where(ad < 1.0, 0.5 * ad * ad, ad - 0.5)


_SPP = 2


def _body(tgt_ref, loc_ref, cnf_ref, reg_ref, pri_ref,
          out_l, out_c, out_r, out_n, bti_sa):
    b = pl.program_id(0)
    f32 = jnp.float32
    S = range(_SPP)

    pcx = pri_ref[0]
    pcy = pri_ref[1]
    pw = pri_ref[2]
    ph = pri_ref[3]
    px0 = pcx - pw * 0.5
    py0 = pcy - ph * 0.5
    px1 = pcx + pw * 0.5
    py1 = pcy + ph * 0.5
    area_p = (px1 - px0) * (py1 - py0)
    row = jax.lax.broadcasted_iota(jnp.int32, (_R, _C), 0)
    col = jax.lax.broadcasted_iota(jnp.int32, (_R, _C), 1)
    idx = row * _C + col
    valid = idx < _NP


    bto = [jnp.full((_R, _C), -1.0, f32) for _ in S]
    bti = [jnp.zeros((_R, _C), jnp.int32) for _ in S]
    bpi = [[] for _ in S]
    for j in range(_NOBJ):
        for s in S:
            tx0 = tgt_ref[0, s, j, 0]
            ty0 = tgt_ref[0, s, j, 1]
            tx1 = tgt_ref[0, s, j, 2]
            ty1 = tgt_ref[0, s, j, 3]
            area_t = (tx1 - tx0) * (ty1 - ty0)
            iw = jnp.maximum(jnp.minimum(tx1, px1) - jnp.maximum(tx0, px0),
                             0.0)
            ih = jnp.maximum(jnp.minimum(ty1, py1) - jnp.maximum(ty0, py0),
                             0.0)
            inter = iw * ih
            ov = inter / (area_t + area_p - inter)
            upd = ov > bto[s]
            bto[s] = jnp.where(upd, ov, bto[s])
            bti[s] = jnp.where(upd, j, bti[s])
            m = jnp.max(ov)
            bpi[s].append(jnp.min(jnp.where(ov == m, idx, _PP)))

    for s in S:
        bti_sa[s] = bti[s]
    colv = jax.lax.broadcasted_iota(jnp.int32, (1, _C), 1)
    for j in range(_NOBJ):
        for s in S:
            r = bpi[s][j] // _C
            c = bpi[s][j] % _C
            rowv = bti_sa[s, pl.ds(r, 1), :]
            bti_sa[s, pl.ds(r, 1), :] = jnp.where(colv == c, j + 64, rowv)
    btim = [None] * _SPP
    forced = [None] * _SPP
    for s in S:
        btif = bti_sa[s]
        forced[s] = btif >= 64
        btim[s] = btif & 63

    mcx = [jnp.zeros((_R, _C), f32) for _ in S]
    mcy = [jnp.zeros((_R, _C), f32) for _ in S]
    lbl = [jnp.zeros((_R, _C), f32) for _ in S]
    rgt = [jnp.zeros((_R, _C), f32) for _ in S]
    for j in range(_NOBJ):
        for s in S:
            mj = btim[s] == j
            mcx[s] = jnp.where(
                mj, (tgt_ref[0, s, j, 0] + tgt_ref[0, s, j, 2]) * 0.5, mcx[s])
            mcy[s] = jnp.where(
                mj, (tgt_ref[0, s, j, 1] + tgt_ref[0, s, j, 3]) * 0.5, mcy[s])
            lbl[s] = jnp.where(mj, tgt_ref[0, s, j, 4], lbl[s])
            rgt[s] = jnp.where(mj, tgt_ref[0, s, j, 5], rgt[s])

    pos = [None] * _SPP
    posf = [None] * _SPP
    conf = [None] * _SPP
    num_pos = [None] * _SPP
    loss_l = [None] * _SPP
    loss_r = [None] * _SPP
    cls = [None] * _SPP
    bits = [None] * _SPP
    num_neg = [None] * _SPP
    for s in S:
        conf[s] = jnp.where(jnp.logical_or(forced[s], bto[s] >= _TH),
                            lbl[s].astype(jnp.int32) + 1, 0)
        pos[s] = conf[s] > 0
        posf[s] = pos[s].astype(f32)
        num_pos[s] = jnp.sum(posf[s])

        lt0 = (mcx[s] - pcx) / (_VAR * pw)
        lt1 = (mcy[s] - pcy) / (_VAR * ph)
        loss_l[s] = jnp.sum((_smooth_l1(loc_ref[0, s, 0] - lt0)
                             + _smooth_l1(loc_ref[0, s, 1] - lt1)) * posf[s])
        loss_r[s] = jnp.sum(_smooth_l1(reg_ref[0, s, 0] - rgt[s]) * posf[s])

        c0 = cnf_ref[0, s, 0]
        c1 = cnf_ref[0, s, 1]
        c2 = cnf_ref[0, s, 2]
        c3 = cnf_ref[0, s, 3]
        cls[s] = (c0, c1, c2, c3)
        cm = jnp.maximum(jnp.maximum(c0, c1), jnp.maximum(c2, c3))
        lse = cm + jnp.log(jnp.exp(c0 - cm) + jnp.exp(c1 - cm)
                           + jnp.exp(c2 - cm) + jnp.exp(c3 - cm))
        g = jnp.where(conf[s] == 0, c0, 0.0) \
            + jnp.where(conf[s] == 1, c1, 0.0) \
            + jnp.where(conf[s] == 2, c2, 0.0) \
            + jnp.where(conf[s] == 3, c3, 0.0)
        lca = lse - g
        lca = jnp.where(pos[s], 0.0, lca)
        lca = jnp.where(valid, lca, -1.0)
        bits[s] = jax.lax.bitcast_convert_type(lca, jnp.int32)
        num_neg[s] = jnp.minimum(_NEGPOS * num_pos[s].astype(jnp.int32),
                                 _NP - 1)

    thr = [jnp.int32(0) for _ in S]
    for k in range(30, -1, -1):
        for s in S:
            cand = thr[s] | jnp.int32(1 << k)
            cnt = jnp.sum(jnp.where(bits[s] >= cand, 1, 0))
            thr[s] = jnp.where(cnt >= num_neg[s], cand, thr[s])

    loss_c = [None] * _SPP
    for s in S:
        neg = bits[s] >= thr[s]
        self32 = jnp.logical_or(pos[s], neg).astype(f32)
        bce = jnp.zeros((_R, _C), f32)
        for k, ck in enumerate(cls[s]):
            st = jnp.where(conf[s] == k, 1.0 - _ALPHA, 0.0) + _ALPHA / 4.0
            bce = bce + (jnp.maximum(ck, 0.0) - ck * st
                         + jnp.log1p(jnp.exp(-jnp.abs(ck))))
        loss_c[s] = jnp.sum(bce * self32)

    acc = [sum(loss_l), sum(loss_c), sum(loss_r), sum(num_pos)]

    @pl.when(b == 0)
    def _():
        out_l[...] = jnp.zeros((8, 128), f32)
        out_c[...] = jnp.zeros((8, 128), f32)
        out_r[...] = jnp.zeros((8, 128), f32)
        out_n[...] = jnp.zeros((8, 128), f32)

    out_l[...] = out_l[...] + acc[0]
    out_c[...] = out_c[...] + acc[1]
    out_r[...] = out_r[...] + acc[2]
    out_n[...] = out_n[...] + acc[3]


def kernel(loc_data, cnf_data, reg_data, targets, priors):
    bs = loc_data.shape[0]
    pad = _PP - _NP
    f32 = jnp.float32

    pad_pri = jnp.broadcast_to(
        jnp.array([-5.0, -5.0, 1e-3, 1e-3], f32), (pad, 4))
    pri = jnp.concatenate([priors.astype(f32), pad_pri], axis=0)
    pri = pri.T.reshape(4, _R, _C)

    def pad_t(x):
        x = jnp.pad(x, ((0, 0), (0, pad), (0, 0)))
        return x.transpose(0, 2, 1).reshape(bs, x.shape[2], _R, _C)

    loc = pad_t(loc_data)
    cnf = pad_t(cnf_data)
    reg = pad_t(reg_data)
    tgt = targets.reshape(bs, _NOBJ, 6)

    loc = loc.reshape(bs // _SPP, _SPP, 2, _R, _C)
    cnf = cnf.reshape(bs // _SPP, _SPP, _NCLS, _R, _C)
    reg = reg.reshape(bs // _SPP, _SPP, 1, _R, _C)
    tgt = tgt.reshape(bs // _SPP, _SPP, _NOBJ, 6)

    out_shape = [jax.ShapeDtypeStruct((8, 128), f32)] * 4
    outs = pl.pallas_call(
        _body,
        grid=(bs // _SPP,),
        in_specs=[
            pl.BlockSpec((1, _SPP, _NOBJ, 6), lambda b: (b, 0, 0, 0),
                         memory_space=pltpu.SMEM),
            pl.BlockSpec((1, _SPP, 2, _R, _C), lambda b: (b, 0, 0, 0, 0)),
            pl.BlockSpec((1, _SPP, _NCLS, _R, _C), lambda b: (b, 0, 0, 0, 0)),
            pl.BlockSpec((1, _SPP, 1, _R, _C), lambda b: (b, 0, 0, 0, 0)),
            pl.BlockSpec((4, _R, _C), lambda b: (0, 0, 0)),
        ],
        out_specs=[pl.BlockSpec((8, 128), lambda b: (0, 0))] * 4,
        out_shape=out_shape,
        scratch_shapes=[pltpu.VMEM((_SPP, _R, _C), jnp.int32)],
    )(tgt, loc, cnf, reg, pri)

    l, c, r, n = [o[0, 0] for o in outs]
    return (l / n, c / n, r / n)


# speedup vs baseline: 1.0947x; 1.0206x over previous
import jax
import jax.numpy as jnp
from jax.experimental import pallas as pl
from jax.experimental.pallas import tpu as pltpu

_NP = 20000
_R, _C = 160, 128
_PP = _R * _C
_NCLS = 4
_NOBJ = 50
_TH = 0.5
_NEGPOS = 3
_VAR = 0.1
_ALPHA = 0.1


def _smooth_l1(d):
    ad = jnp.abs(d)
    return jnp.where(ad < 1.0, 0.5 * ad * ad, ad - 0.5)


_SPP = 2


def _one_sample(tgt_ref, loc_ref, cnf_ref, reg_ref, s, pri):
    f32 = jnp.float32
    pcx, pcy, pw, ph, px0, py0, px1, py1, area_p, idx, valid = pri

    bto = jnp.full((_R, _C), -1.0, f32)
    bti = jnp.zeros((_R, _C), jnp.int32)
    bpi = []
    for j in range(_NOBJ):
        tx0 = tgt_ref[0, s, j, 0]
        ty0 = tgt_ref[0, s, j, 1]
        tx1 = tgt_ref[0, s, j, 2]
        ty1 = tgt_ref[0, s, j, 3]
        area_t = (tx1 - tx0) * (ty1 - ty0)
        iw = jnp.maximum(jnp.minimum(tx1, px1) - jnp.maximum(tx0, px0), 0.0)
        ih = jnp.maximum(jnp.minimum(ty1, py1) - jnp.maximum(ty0, py0), 0.0)
        inter = iw * ih
        ov = inter / (area_t + area_p - inter)
        upd = ov > bto
        bto = jnp.where(upd, ov, bto)
        bti = jnp.where(upd, j, bti)
        m = jnp.max(ov)
        bpi.append(jnp.min(jnp.where(ov == m, idx, _PP)))

    for j in range(_NOBJ):
        hit = idx == bpi[j]
        bto = jnp.where(hit, 2.0, bto)
        bti = jnp.where(hit, j, bti)

    mcx = jnp.zeros((_R, _C), f32)
    mcy = jnp.zeros((_R, _C), f32)
    lbl = jnp.zeros((_R, _C), f32)
    rgt = jnp.zeros((_R, _C), f32)
    for j in range(_NOBJ):
        mj = bti == j
        mcx = jnp.where(mj, (tgt_ref[0, s, j, 0] + tgt_ref[0, s, j, 2]) * 0.5, mcx)
        mcy = jnp.where(mj, (tgt_ref[0, s, j, 1] + tgt_ref[0, s, j, 3]) * 0.5, mcy)
        lbl = jnp.where(mj, tgt_ref[0, s, j, 4], lbl)
        rgt = jnp.where(mj, tgt_ref[0, s, j, 5], rgt)

    conf = jnp.where(bto < _TH, 0, lbl.astype(jnp.int32) + 1)
    pos = conf > 0
    posf = pos.astype(f32)
    num_pos = jnp.sum(posf)

    lt0 = (mcx - pcx) / (_VAR * pw)
    lt1 = (mcy - pcy) / (_VAR * ph)
    loss_l = jnp.sum((_smooth_l1(loc_ref[0, s, 0] - lt0)
                      + _smooth_l1(loc_ref[0, s, 1] - lt1)) * posf)

    loss_r = jnp.sum(_smooth_l1(reg_ref[0, s, 0] - rgt) * posf)

    c0 = cnf_ref[0, s, 0]
    c1 = cnf_ref[0, s, 1]
    c2 = cnf_ref[0, s, 2]
    c3 = cnf_ref[0, s, 3]
    cm = jnp.maximum(jnp.maximum(c0, c1), jnp.maximum(c2, c3))
    lse = cm + jnp.log(jnp.exp(c0 - cm) + jnp.exp(c1 - cm)
                       + jnp.exp(c2 - cm) + jnp.exp(c3 - cm))
    g = jnp.where(conf == 0, c0, 0.0) + jnp.where(conf == 1, c1, 0.0) \
        + jnp.where(conf == 2, c2, 0.0) + jnp.where(conf == 3, c3, 0.0)
    lca = lse - g
    lca = jnp.where(pos, 0.0, lca)
    lca = jnp.where(valid, lca, -1.0)

    bits = jax.lax.bitcast_convert_type(lca, jnp.int32)
    num_neg = jnp.minimum(_NEGPOS * num_pos.astype(jnp.int32), _NP - 1)
    thr = jnp.int32(0)
    for k in range(30, -1, -1):
        cand = thr | jnp.int32(1 << k)
        cnt = jnp.sum(jnp.where(bits >= cand, 1, 0))
        thr = jnp.where(cnt >= num_neg, cand, thr)
    neg = bits >= thr
    self32 = jnp.logical_or(pos, neg).astype(f32)

    bce = jnp.zeros((_R, _C), f32)
    for k, ck in enumerate((c0, c1, c2, c3)):
        st = jnp.where(conf == k, 1.0 - _ALPHA, 0.0) + _ALPHA / 4.0
        bce = bce + (jnp.maximum(ck, 0.0) - ck * st
                     + jnp.log1p(jnp.exp(-jnp.abs(ck))))
    loss_c = jnp.sum(bce * self32)

    return loss_l, loss_c, loss_r, num_pos


def _body(tgt_ref, loc_ref, cnf_ref, reg_ref, pri_ref,
          out_l, out_c, out_r, out_n):
    b = pl.program_id(0)
    f32 = jnp.float32

    pcx = pri_ref[0]
    pcy = pri_ref[1]
    pw = pri_ref[2]
    ph = pri_ref[3]
    px0 = pcx - pw * 0.5
    py0 = pcy - ph * 0.5
    px1 = pcx + pw * 0.5
    py1 = pcy + ph * 0.5
    area_p = (px1 - px0) * (py1 - py0)
    row = jax.lax.broadcasted_iota(jnp.int32, (_R, _C), 0)
    col = jax.lax.broadcasted_iota(jnp.int32, (_R, _C), 1)
    idx = row * _C + col
    valid = idx < _NP
    pri = (pcx, pcy, pw, ph, px0, py0, px1, py1, area_p, idx, valid)

    acc = [jnp.float32(0.0)] * 4
    for s in range(_SPP):
        res = _one_sample(tgt_ref, loc_ref, cnf_ref, reg_ref, s, pri)
        acc = [a + r for a, r in zip(acc, res)]

    @pl.when(b == 0)
    def _():
        out_l[...] = jnp.zeros((8, 128), f32)
        out_c[...] = jnp.zeros((8, 128), f32)
        out_r[...] = jnp.zeros((8, 128), f32)
        out_n[...] = jnp.zeros((8, 128), f32)

    out_l[...] = out_l[...] + acc[0]
    out_c[...] = out_c[...] + acc[1]
    out_r[...] = out_r[...] + acc[2]
    out_n[...] = out_n[...] + acc[3]


def kernel(loc_data, cnf_data, reg_data, targets, priors):
    bs = loc_data.shape[0]
    pad = _PP - _NP
    f32 = jnp.float32

    pad_pri = jnp.broadcast_to(
        jnp.array([-5.0, -5.0, 1e-3, 1e-3], f32), (pad, 4))
    pri = jnp.concatenate([priors.astype(f32), pad_pri], axis=0)
    pri = pri.T.reshape(4, _R, _C)

    def pad_t(x):
        x = jnp.pad(x, ((0, 0), (0, pad), (0, 0)))
        return x.transpose(0, 2, 1).reshape(bs, x.shape[2], _R, _C)

    loc = pad_t(loc_data)
    cnf = pad_t(cnf_data)
    reg = pad_t(reg_data)
    tgt = targets.reshape(bs, _NOBJ, 6)

    loc = loc.reshape(bs // _SPP, _SPP, 2, _R, _C)
    cnf = cnf.reshape(bs // _SPP, _SPP, _NCLS, _R, _C)
    reg = reg.reshape(bs // _SPP, _SPP, 1, _R, _C)
    tgt = tgt.reshape(bs // _SPP, _SPP, _NOBJ, 6)

    out_shape = [jax.ShapeDtypeStruct((8, 128), f32)] * 4
    outs = pl.pallas_call(
        _body,
        grid=(bs // _SPP,),
        in_specs=[
            pl.BlockSpec((1, _SPP, _NOBJ, 6), lambda b: (b, 0, 0, 0),
                         memory_space=pltpu.SMEM),
            pl.BlockSpec((1, _SPP, 2, _R, _C), lambda b: (b, 0, 0, 0, 0)),
            pl.BlockSpec((1, _SPP, _NCLS, _R, _C), lambda b: (b, 0, 0, 0, 0)),
            pl.BlockSpec((1, _SPP, 1, _R, _C), lambda b: (b, 0, 0, 0, 0)),
            pl.BlockSpec((4, _R, _C), lambda b: (0, 0, 0)),
        ],
        out_specs=[pl.BlockSpec((8, 128), lambda b: (0, 0))] * 4,
        out_shape=out_shape,
    )(tgt, loc, cnf, reg, pri)

    l, c, r, n = [o[0, 0] for o in outs]
    return (l / n, c / n, r / n)


# speedup vs baseline: 1.2058x; 1.1015x over previous
import jax
import jax.numpy as jnp
from jax.experimental import pallas as pl
from jax.experimental.pallas import tpu as pltpu

_NP = 20000
_R, _C = 160, 128
_PP = _R * _C
_NCLS = 4
_NOBJ = 50
_TH = 0.5
_NEGPOS = 3
_VAR = 0.1
_ALPHA = 0.1


def _smooth_l1(d):
    ad = jnp.abs(d)
    return jnp.where(ad < 1.0, 0.5 * ad * ad, ad - 0.5)


_SPP = 2


def _one_sample(tgt_ref, loc_ref, cnf_ref, reg_ref, s, pri):
    f32 = jnp.float32
    pcx, pcy, pw, ph, px0, py0, px1, py1, area_p, idx, valid = pri

    bto = jnp.full((_R, _C), -1.0, f32)
    bti = jnp.zeros((_R, _C), jnp.int32)
    bpi = []
    for j in range(_NOBJ):
        tx0 = tgt_ref[0, s, j, 0]
        ty0 = tgt_ref[0, s, j, 1]
        tx1 = tgt_ref[0, s, j, 2]
        ty1 = tgt_ref[0, s, j, 3]
        area_t = (tx1 - tx0) * (ty1 - ty0)
        iw = jnp.maximum(jnp.minimum(tx1, px1) - jnp.maximum(tx0, px0), 0.0)
        ih = jnp.maximum(jnp.minimum(ty1, py1) - jnp.maximum(ty0, py0), 0.0)
        inter = iw * ih
        ov = inter / (area_t + area_p - inter)
        upd = ov > bto
        bto = jnp.where(upd, ov, bto)
        bti = jnp.where(upd, j, bti)
        m = jnp.max(ov)
        bpi.append(jnp.min(jnp.where(ov == m, idx, _PP)))

    for j in range(_NOBJ):
        hit = idx == bpi[j]
        bto = jnp.where(hit, 2.0, bto)
        bti = jnp.where(hit, j, bti)

    mcx = jnp.zeros((_R, _C), f32)
    mcy = jnp.zeros((_R, _C), f32)
    lbl = jnp.zeros((_R, _C), f32)
    rgt = jnp.zeros((_R, _C), f32)
    for j in range(_NOBJ):
        mj = bti == j
        mcx = jnp.where(mj, (tgt_ref[0, s, j, 0] + tgt_ref[0, s, j, 2]) * 0.5, mcx)
        mcy = jnp.where(mj, (tgt_ref[0, s, j, 1] + tgt_ref[0, s, j, 3]) * 0.5, mcy)
        lbl = jnp.where(mj, tgt_ref[0, s, j, 4], lbl)
        rgt = jnp.where(mj, tgt_ref[0, s, j, 5], rgt)

    conf = jnp.where(bto < _TH, 0, lbl.astype(jnp.int32) + 1)
    pos = conf > 0
    posf = pos.astype(f32)
    num_pos = jnp.sum(posf)

    lt0 = (mcx - pcx) / (_VAR * pw)
    lt1 = (mcy - pcy) / (_VAR * ph)
    loss_l = jnp.sum((_smooth_l1(loc_ref[0, s, 0] - lt0)
                      + _smooth_l1(loc_ref[0, s, 1] - lt1)) * posf)

    loss_r = jnp.sum(_smooth_l1(reg_ref[0, s, 0] - rgt) * posf)

    c0 = cnf_ref[0, s, 0]
    c1 = cnf_ref[0, s, 1]
    c2 = cnf_ref[0, s, 2]
    c3 = cnf_ref[0, s, 3]
    cm = jnp.maximum(jnp.maximum(c0, c1), jnp.maximum(c2, c3))
    lse = cm + jnp.log(jnp.exp(c0 - cm) + jnp.exp(c1 - cm)
                       + jnp.exp(c2 - cm) + jnp.exp(c3 - cm))
    g = jnp.where(conf == 0, c0, 0.0) + jnp.where(conf == 1, c1, 0.0) \
        + jnp.where(conf == 2, c2, 0.0) + jnp.where(conf == 3, c3, 0.0)
    lca = lse - g
    lca = jnp.where(pos, 0.0, lca)
    lca = jnp.where(valid, lca, -1.0)

    bits = jax.lax.bitcast_convert_type(lca, jnp.int32)
    num_neg = jnp.minimum(_NEGPOS * num_pos.astype(jnp.int32), _NP - 1)
    thr = jnp.int32(0)
    for b1 in range(30, 0, -2):
        t1 = thr | jnp.int32(1 << b1)
        t2 = t1 | jnp.int32(1 << (b1 - 1))
        t3 = thr | jnp.int32(1 << (b1 - 1))
        n1 = jnp.sum(jnp.where(bits >= t1, 1, 0))
        n2 = jnp.sum(jnp.where(bits >= t2, 1, 0))
        n3 = jnp.sum(jnp.where(bits >= t3, 1, 0))
        thr = jnp.where(n1 >= num_neg,
                        jnp.where(n2 >= num_neg, t2, t1),
                        jnp.where(n3 >= num_neg, t3, thr))
    cand = thr | jnp.int32(1)
    cnt = jnp.sum(jnp.where(bits >= cand, 1, 0))
    thr = jnp.where(cnt >= num_neg, cand, thr)
    neg = bits >= thr
    self32 = jnp.logical_or(pos, neg).astype(f32)

    bce = jnp.zeros((_R, _C), f32)
    for k, ck in enumerate((c0, c1, c2, c3)):
        st = jnp.where(conf == k, 1.0 - _ALPHA, 0.0) + _ALPHA / 4.0
        bce = bce + (jnp.maximum(ck, 0.0) - ck * st
                     + jnp.log1p(jnp.exp(-jnp.abs(ck))))
    loss_c = jnp.sum(bce * self32)

    return loss_l, loss_c, loss_r, num_pos


def _body(tgt_ref, loc_ref, cnf_ref, reg_ref, pri_ref,
          out_l, out_c, out_r, out_n):
    b = pl.program_id(0)
    f32 = jnp.float32

    pcx = pri_ref[0]
    pcy = pri_ref[1]
    pw = pri_ref[2]
    ph = pri_ref[3]
    px0 = pcx - pw * 0.5
    py0 = pcy - ph * 0.5
    px1 = pcx + pw * 0.5
    py1 = pcy + ph * 0.5
    area_p = (px1 - px0) * (py1 - py0)
    row = jax.lax.broadcasted_iota(jnp.int32, (_R, _C), 0)
    col = jax.lax.broadcasted_iota(jnp.int32, (_R, _C), 1)
    idx = row * _C + col
    valid = idx < _NP
    pri = (pcx, pcy, pw, ph, px0, py0, px1, py1, area_p, idx, valid)

    acc = [jnp.float32(0.0)] * 4
    for s in range(_SPP):
        res = _one_sample(tgt_ref, loc_ref, cnf_ref, reg_ref, s, pri)
        acc = [a + r for a, r in zip(acc, res)]

    @pl.when(b == 0)
    def _():
        out_l[...] = jnp.zeros((8, 128), f32)
        out_c[...] = jnp.zeros((8, 128), f32)
        out_r[...] = jnp.zeros((8, 128), f32)
        out_n[...] = jnp.zeros((8, 128), f32)

    out_l[...] = out_l[...] + acc[0]
    out_c[...] = out_c[...] + acc[1]
    out_r[...] = out_r[...] + acc[2]
    out_n[...] = out_n[...] + acc[3]


def kernel(loc_data, cnf_data, reg_data, targets, priors):
    bs = loc_data.shape[0]
    pad = _PP - _NP
    f32 = jnp.float32

    pad_pri = jnp.broadcast_to(
        jnp.array([-5.0, -5.0, 1e-3, 1e-3], f32), (pad, 4))
    pri = jnp.concatenate([priors.astype(f32), pad_pri], axis=0)
    pri = pri.T.reshape(4, _R, _C)

    def pad_t(x):
        x = jnp.pad(x, ((0, 0), (0, pad), (0, 0)))
        return x.transpose(0, 2, 1).reshape(bs, x.shape[2], _R, _C)

    loc = pad_t(loc_data)
    cnf = pad_t(cnf_data)
    reg = pad_t(reg_data)
    tgt = targets.reshape(bs, _NOBJ, 6)

    loc = loc.reshape(bs // _SPP, _SPP, 2, _R, _C)
    cnf = cnf.reshape(bs // _SPP, _SPP, _NCLS, _R, _C)
    reg = reg.reshape(bs // _SPP, _SPP, 1, _R, _C)
    tgt = tgt.reshape(bs // _SPP, _SPP, _NOBJ, 6)

    out_shape = [jax.ShapeDtypeStruct((8, 128), f32)] * 4
    outs = pl.pallas_call(
        _body,
        grid=(bs // _SPP,),
        in_specs=[
            pl.BlockSpec((1, _SPP, _NOBJ, 6), lambda b: (b, 0, 0, 0),
                         memory_space=pltpu.SMEM),
            pl.BlockSpec((1, _SPP, 2, _R, _C), lambda b: (b, 0, 0, 0, 0)),
            pl.BlockSpec((1, _SPP, _NCLS, _R, _C), lambda b: (b, 0, 0, 0, 0)),
            pl.BlockSpec((1, _SPP, 1, _R, _C), lambda b: (b, 0, 0, 0, 0)),
            pl.BlockSpec((4, _R, _C), lambda b: (0, 0, 0)),
        ],
        out_specs=[pl.BlockSpec((8, 128), lambda b: (0, 0))] * 4,
        out_shape=out_shape,
    )(tgt, loc, cnf, reg, pri)

    l, c, r, n = [o[0, 0] for o in outs]
    return (l / n, c / n, r / n)


# speedup vs baseline: 1.2351x; 1.0242x over previous
import jax
import jax.numpy as jnp
from jax.experimental import pallas as pl
from jax.experimental.pallas import tpu as pltpu

_NP = 20000
_R, _C = 160, 128
_PP = _R * _C
_NCLS = 4
_NOBJ = 50
_TH = 0.5
_NEGPOS = 3
_VAR = 0.1
_ALPHA = 0.1


def _smooth_l1(d):
    ad = jnp.abs(d)
    return jnp.where(ad < 1.0, 0.5 * ad * ad, ad - 0.5)


_SPP = 2


def _one_sample(tgt_ref, loc_ref, cnf_ref, reg_ref, s, pri):
    f32 = jnp.float32
    pcx, pcy, pw, ph, px0, py0, px1, py1, area_p, idx, valid = pri

    bto = jnp.full((_R, _C), -1.0, f32)
    bti = jnp.zeros((_R, _C), jnp.int32)
    bpi = []
    for j in range(_NOBJ):
        tx0 = tgt_ref[0, s, j, 0]
        ty0 = tgt_ref[0, s, j, 1]
        tx1 = tgt_ref[0, s, j, 2]
        ty1 = tgt_ref[0, s, j, 3]
        area_t = (tx1 - tx0) * (ty1 - ty0)
        iw = jnp.maximum(jnp.minimum(tx1, px1) - jnp.maximum(tx0, px0), 0.0)
        ih = jnp.maximum(jnp.minimum(ty1, py1) - jnp.maximum(ty0, py0), 0.0)
        inter = iw * ih
        ov = inter / (area_t + area_p - inter)
        upd = ov > bto
        bto = jnp.where(upd, ov, bto)
        bti = jnp.where(upd, j, bti)
        m = jnp.max(ov)
        bpi.append(jnp.min(jnp.where(ov == m, idx, _PP)))

    for j in range(_NOBJ):
        hit = idx == bpi[j]
        bto = jnp.where(hit, 2.0, bto)
        bti = jnp.where(hit, j, bti)

    mcx = jnp.zeros((_R, _C), f32)
    mcy = jnp.zeros((_R, _C), f32)
    lbl = jnp.zeros((_R, _C), f32)
    rgt = jnp.zeros((_R, _C), f32)
    for j in range(_NOBJ):
        mj = bti == j
        mcx = jnp.where(mj, (tgt_ref[0, s, j, 0] + tgt_ref[0, s, j, 2]) * 0.5, mcx)
        mcy = jnp.where(mj, (tgt_ref[0, s, j, 1] + tgt_ref[0, s, j, 3]) * 0.5, mcy)
        lbl = jnp.where(mj, tgt_ref[0, s, j, 4], lbl)
        rgt = jnp.where(mj, tgt_ref[0, s, j, 5], rgt)

    conf = jnp.where(bto < _TH, 0, lbl.astype(jnp.int32) + 1)
    pos = conf > 0
    posf = pos.astype(f32)
    num_pos = jnp.sum(posf)

    lt0 = (mcx - pcx) / (_VAR * pw)
    lt1 = (mcy - pcy) / (_VAR * ph)
    loss_l = jnp.sum((_smooth_l1(loc_ref[0, s, 0] - lt0)
                      + _smooth_l1(loc_ref[0, s, 1] - lt1)) * posf)

    loss_r = jnp.sum(_smooth_l1(reg_ref[0, s, 0] - rgt) * posf)

    c0 = cnf_ref[0, s, 0]
    c1 = cnf_ref[0, s, 1]
    c2 = cnf_ref[0, s, 2]
    c3 = cnf_ref[0, s, 3]
    cm = jnp.maximum(jnp.maximum(c0, c1), jnp.maximum(c2, c3))
    lse = cm + jnp.log(jnp.exp(c0 - cm) + jnp.exp(c1 - cm)
                       + jnp.exp(c2 - cm) + jnp.exp(c3 - cm))
    g = jnp.where(conf == 0, c0, 0.0) + jnp.where(conf == 1, c1, 0.0) \
        + jnp.where(conf == 2, c2, 0.0) + jnp.where(conf == 3, c3, 0.0)
    lca = lse - g
    lca = jnp.where(pos, 0.0, lca)
    lca = jnp.where(valid, lca, -1.0)

    bits = jax.lax.bitcast_convert_type(lca, jnp.int32)
    num_neg = jnp.minimum(_NEGPOS * num_pos.astype(jnp.int32), _NP - 1)
    thr = jnp.int32(0)
    for b2 in range(30, 2, -3):
        t100 = thr | jnp.int32(1 << b2)
        t010 = thr | jnp.int32(1 << (b2 - 1))
        t001 = thr | jnp.int32(1 << (b2 - 2))
        t110 = t100 | jnp.int32(1 << (b2 - 1))
        t101 = t100 | jnp.int32(1 << (b2 - 2))
        t011 = t010 | jnp.int32(1 << (b2 - 2))
        t111 = t110 | jnp.int32(1 << (b2 - 2))
        n100 = jnp.sum(jnp.where(bits >= t100, 1, 0))
        n010 = jnp.sum(jnp.where(bits >= t010, 1, 0))
        n001 = jnp.sum(jnp.where(bits >= t001, 1, 0))
        n110 = jnp.sum(jnp.where(bits >= t110, 1, 0))
        n101 = jnp.sum(jnp.where(bits >= t101, 1, 0))
        n011 = jnp.sum(jnp.where(bits >= t011, 1, 0))
        n111 = jnp.sum(jnp.where(bits >= t111, 1, 0))
        nn = num_neg
        thr = jnp.where(
            n100 >= nn,
            jnp.where(n110 >= nn,
                      jnp.where(n111 >= nn, t111, t110),
                      jnp.where(n101 >= nn, t101, t100)),
            jnp.where(n010 >= nn,
                      jnp.where(n011 >= nn, t011, t010),
                      jnp.where(n001 >= nn, t001, thr)))
    cand = thr | jnp.int32(1)
    cnt = jnp.sum(jnp.where(bits >= cand, 1, 0))
    thr = jnp.where(cnt >= num_neg, cand, thr)
    neg = bits >= thr
    self32 = jnp.logical_or(pos, neg).astype(f32)

    bce = jnp.zeros((_R, _C), f32)
    for k, ck in enumerate((c0, c1, c2, c3)):
        st = jnp.where(conf == k, 1.0 - _ALPHA, 0.0) + _ALPHA / 4.0
        bce = bce + (jnp.maximum(ck, 0.0) - ck * st
                     + jnp.log1p(jnp.exp(-jnp.abs(ck))))
    loss_c = jnp.sum(bce * self32)

    return loss_l, loss_c, loss_r, num_pos


def _body(tgt_ref, loc_ref, cnf_ref, reg_ref, pri_ref,
          out_l, out_c, out_r, out_n):
    b = pl.program_id(0)
    f32 = jnp.float32

    pcx = pri_ref[0]
    pcy = pri_ref[1]
    pw = pri_ref[2]
    ph = pri_ref[3]
    px0 = pcx - pw * 0.5
    py0 = pcy - ph * 0.5
    px1 = pcx + pw * 0.5
    py1 = pcy + ph * 0.5
    area_p = (px1 - px0) * (py1 - py0)
    row = jax.lax.broadcasted_iota(jnp.int32, (_R, _C), 0)
    col = jax.lax.broadcasted_iota(jnp.int32, (_R, _C), 1)
    idx = row * _C + col
    valid = idx < _NP
    pri = (pcx, pcy, pw, ph, px0, py0, px1, py1, area_p, idx, valid)

    acc = [jnp.float32(0.0)] * 4
    for s in range(_SPP):
        res = _one_sample(tgt_ref, loc_ref, cnf_ref, reg_ref, s, pri)
        acc = [a + r for a, r in zip(acc, res)]

    @pl.when(b == 0)
    def _():
        out_l[...] = jnp.zeros((8, 128), f32)
        out_c[...] = jnp.zeros((8, 128), f32)
        out_r[...] = jnp.zeros((8, 128), f32)
        out_n[...] = jnp.zeros((8, 128), f32)

    out_l[...] = out_l[...] + acc[0]
    out_c[...] = out_c[...] + acc[1]
    out_r[...] = out_r[...] + acc[2]
    out_n[...] = out_n[...] + acc[3]


def kernel(loc_data, cnf_data, reg_data, targets, priors):
    bs = loc_data.shape[0]
    pad = _PP - _NP
    f32 = jnp.float32

    pad_pri = jnp.broadcast_to(
        jnp.array([-5.0, -5.0, 1e-3, 1e-3], f32), (pad, 4))
    pri = jnp.concatenate([priors.astype(f32), pad_pri], axis=0)
    pri = pri.T.reshape(4, _R, _C)

    def pad_t(x):
        x = jnp.pad(x, ((0, 0), (0, pad), (0, 0)))
        return x.transpose(0, 2, 1).reshape(bs, x.shape[2], _R, _C)

    loc = pad_t(loc_data)
    cnf = pad_t(cnf_data)
    reg = pad_t(reg_data)
    tgt = targets.reshape(bs, _NOBJ, 6)

    loc = loc.reshape(bs // _SPP, _SPP, 2, _R, _C)
    cnf = cnf.reshape(bs // _SPP, _SPP, _NCLS, _R, _C)
    reg = reg.reshape(bs // _SPP, _SPP, 1, _R, _C)
    tgt = tgt.reshape(bs // _SPP, _SPP, _NOBJ, 6)

    out_shape = [jax.ShapeDtypeStruct((8, 128), f32)] * 4
    outs = pl.pallas_call(
        _body,
        grid=(bs // _SPP,),
        in_specs=[
            pl.BlockSpec((1, _SPP, _NOBJ, 6), lambda b: (b, 0, 0, 0),
                         memory_space=pltpu.SMEM),
            pl.BlockSpec((1, _SPP, 2, _R, _C), lambda b: (b, 0, 0, 0, 0)),
            pl.BlockSpec((1, _SPP, _NCLS, _R, _C), lambda b: (b, 0, 0, 0, 0)),
            pl.BlockSpec((1, _SPP, 1, _R, _C), lambda b: (b, 0, 0, 0, 0)),
            pl.BlockSpec((4, _R, _C), lambda b: (0, 0, 0)),
        ],
        out_specs=[pl.BlockSpec((8, 128), lambda b: (0, 0))] * 4,
        out_shape=out_shape,
    )(tgt, loc, cnf, reg, pri)

    l, c, r, n = [o[0, 0] for o in outs]
    return (l / n, c / n, r / n)
